# baseline (speedup 1.0000x reference)
import jax
import jax.numpy as jnp
from jax.experimental import pallas as pl
from jax.experimental.pallas import tpu as pltpu


def _rowdot_kernel(gu_ref, gi_ref, out_ref):
    out_ref[...] = jnp.sum(gu_ref[0] * gi_ref[0], axis=0)


def kernel(inputs):
    n = inputs.shape[1]
    d = inputs.shape[2]
    t = jnp.swapaxes(inputs, 1, 2)
    block = 2048
    return pl.pallas_call(
        _rowdot_kernel,
        grid=(n // block,),
        in_specs=[
            pl.BlockSpec((1, d, block), lambda i: (0, 0, i)),
            pl.BlockSpec((1, d, block), lambda i: (1, 0, i)),
        ],
        out_specs=pl.BlockSpec((block,), lambda i: (i,)),
        out_shape=jax.ShapeDtypeStruct((n,), inputs.dtype),
        compiler_params=pltpu.CompilerParams(
            dimension_semantics=("arbitrary",),
        ),
    )(t, t)


# speedup vs baseline: 1.5407x; 1.5407x over previous
import jax
import jax.numpy as jnp
from jax.experimental import pallas as pl
from jax.experimental.pallas import tpu as pltpu


def _rowdot_kernel(gu_ref, gi_ref, out_ref):
    out_ref[...] = jnp.sum(gu_ref[0] * gi_ref[0], axis=0)


def kernel(inputs):
    n = inputs.shape[1]
    d = inputs.shape[2]
    t = jnp.swapaxes(inputs, 1, 2)
    block = 8192
    return pl.pallas_call(
        _rowdot_kernel,
        grid=(n // block,),
        in_specs=[
            pl.BlockSpec((1, d, block), lambda i: (0, 0, i)),
            pl.BlockSpec((1, d, block), lambda i: (1, 0, i)),
        ],
        out_specs=pl.BlockSpec((block,), lambda i: (i,)),
        out_shape=jax.ShapeDtypeStruct((n,), inputs.dtype),
        compiler_params=pltpu.CompilerParams(
            dimension_semantics=("arbitrary",),
        ),
    )(t, t)
